# baseline (speedup 1.0000x reference)
import functools

import jax
import jax.numpy as jnp
import numpy as np
from jax import lax
from jax.experimental import pallas as pl
from jax.experimental.pallas import tpu as pltpu
from jax.experimental.pallas import tpu_sc as plsc

B, N, DIN, D, H, DH, L, K, DFF, DOUT = 4, 2048, 2, 128, 4, 32, 3, 16, 256, 128
RB = 256
NB = N // RB
NEG = -1e30
EPS = 1e-5
BF = jnp.bfloat16
F32 = jnp.float32

NW = 32
QPW = (B * N) // NW
CH = 16
NCH = QPW // CH


def _bfc(t):
    return t.astype(BF).astype(F32)


def _mm(a, b):
    return jnp.dot(a.astype(BF), b.astype(BF), preferred_element_type=F32)


def _prep_body(pos_ref, posT_ref, h_ref, wp_ref, bp_ref,
               x_ref, idx_ref):
    pos = pos_ref[...]
    posT = posT_ref[...]
    hh = h_ref[...]
    wp = wp_ref[...]
    bp = bp_ref[...]
    x_ref[...] = (_bfc(hh[:, 0:1]) * _bfc(wp[0:1, :])
                  + _bfc(hh[:, 1:2]) * _bfc(wp[1:2, :])) + bp
    px_c, py_c = pos[:, 0:1], pos[:, 1:2]
    px_r, py_r = posT[0:1, :], posT[1:2, :]
    sq_c = px_c * px_c + py_c * py_c
    sq_r = px_r * px_r + py_r * py_r
    mm = _bfc(px_c) * _bfc(px_r) + _bfc(py_c) * _bfc(py_r)
    d2 = (sq_c + sq_r) - 2.0 * mm
    lane = jax.lax.broadcasted_iota(jnp.int32, (RB, N), 1)
    cols = []
    for _ in range(K):
        m = jnp.min(d2, axis=-1, keepdims=True)
        a2 = jnp.min(jnp.where(d2 == m, lane, jnp.int32(N)), axis=-1,
                     keepdims=True)
        d2 = jnp.where(lane == a2, 1e30, d2)
        cols.append(a2)
    idx_ref[...] = jnp.concatenate(cols, axis=1) + pl.program_id(0) * N


def _pack(x):
    bits = lax.bitcast_convert_type(_bfc(x), jnp.int32)
    lo = lax.shift_right_logical(bits[:, :64], jnp.int32(16))
    hi = bits[:, 64:] & jnp.int32(-65536)
    return lax.bitcast_convert_type(lo | hi, F32)


def _qkv_body(x_ref, s_ref, b_ref, wq_ref, wk_ref, wv_ref,
              q_ref, k_ref, v_ref):
    x = x_ref[...]
    m = jnp.mean(x, -1, keepdims=True)
    xc = x - m
    var = jnp.mean(xc * xc, -1, keepdims=True)
    xn = (xc / jnp.sqrt(var + EPS)) * s_ref[...] + b_ref[...]
    q_ref[...] = _pack(_mm(xn, wq_ref[...]))
    k_ref[...] = _pack(_mm(xn, wk_ref[...]))
    v_ref[...] = _pack(_mm(xn, wv_ref[...]))


def _mlp_body(o_ref, x_ref, wo_ref, s_ref, b_ref, w1_ref, b1_ref,
              w2_ref, b2_ref, *rest):
    if len(rest) == 2:
        wout_ref, out_ref = rest
    else:
        wout_ref, (out_ref,) = None, rest
    x = x_ref[...] + _mm(o_ref[...], wo_ref[...])
    m = jnp.mean(x, -1, keepdims=True)
    xc = x - m
    var = jnp.mean(xc * xc, -1, keepdims=True)
    xn = (xc / jnp.sqrt(var + EPS)) * s_ref[...] + b_ref[...]
    hmid = jax.nn.gelu(_mm(xn, w1_ref[...]) + b1_ref[...])
    y = x + (_mm(hmid, w2_ref[...]) + b2_ref[...])
    if wout_ref is not None:
        out_ref[...] = _mm(y, wout_ref[...])
    else:
        out_ref[...] = y


def _rne_bf16(v):
    bits = lax.bitcast_convert_type(v, jnp.int32)
    r = bits + jnp.int32(0x7FFF) + ((bits >> 16) & jnp.int32(1))
    return lax.bitcast_convert_type(r & jnp.int32(-65536), F32)


DP = D // 2


def _unlo(v):
    bits = lax.bitcast_convert_type(v, jnp.int32)
    return lax.bitcast_convert_type(bits << 16, F32)


def _unhi(v):
    bits = lax.bitcast_convert_type(v, jnp.int32)
    return lax.bitcast_convert_type(bits & jnp.int32(-65536), F32)


def _sc_attn_body(kf_ref, vf_ref, q_ref, posx_ref, posy_ref, idx_ref,
                  wrel_ref, o_ref,
                  idxst, qst, posbx, posby, wrelb,
                  knA, vnA, knB, vnB, ob, qtmp, attnb,
                  semA, semB, irdh, l):
    c_id = lax.axis_index("c")
    s_id = lax.axis_index("s")
    wid = s_id * 2 + c_id
    b = wid // 8
    nb0 = (wid % 8) * QPW
    base = b * N

    pltpu.sync_copy(posx_ref.at[b], posbx)
    pltpu.sync_copy(posy_ref.at[b], posby)
    pltpu.sync_copy(wrel_ref, wrelb)
    pltpu.sync_copy(idx_ref.at[b, pl.ds(nb0 * K, QPW * K)], idxst)
    pltpu.sync_copy(q_ref.at[b, pl.ds(nb0, QPW)], qst)

    wv0 = wrelb[pl.ds(0, 16)]
    wv1 = wrelb[pl.ds(16, 16)]

    def _wsel(i):
        return wv0[i] if i < 16 else wv1[i - 16]

    wx = [_wsel(l * 2 * H + hh) for hh in range(H)]
    wy = [_wsel(l * 2 * H + H + hh) for hh in range(H)]

    biota = lax.iota(jnp.int32, 16)
    rowk = [biota * K + k for k in range(K)]

    def fire(c, knb, vnb, sem):
        idxs = idxst.at[pl.ds(c * (CH * K), CH * K)]
        pltpu.async_copy(kf_ref.at[idxs], knb, sem)
        pltpu.async_copy(vf_ref.at[idxs], vnb, sem)

    def wait(c, knb, vnb, sem):
        idxs = idxst.at[pl.ds(c * (CH * K), CH * K)]
        pltpu.make_async_copy(kf_ref.at[idxs], knb, sem).wait()
        pltpu.make_async_copy(vf_ref.at[idxs], vnb, sem).wait()

    def compute(c, knb, vnb):
        n0 = nb0 + c * CH
        qloc = n0 + biota
        posqx = plsc.load_gather(posbx, [qloc])
        posqy = plsc.load_gather(posby, [qloc])
        relx = []
        rely = []
        for k in range(K):
            nk = plsc.load_gather(idxst, [c * (CH * K) + rowk[k]]) - base
            px = plsc.load_gather(posbx, [nk])
            py = plsc.load_gather(posby, [nk])
            relx.append(_rne_bf16(px - posqx))
            rely.append(_rne_bf16(py - posqy))

        qrow = c * CH + biota
        acch = [[None] * K for _ in range(H)]
        for hp in range(2):
            def qpre(dd, carry, hp=hp):
                col = jnp.full((16,), hp * 32 + dd, jnp.int32)
                qpair = plsc.load_gather(qst, [qrow, col])
                plsc.store_scatter(qtmp, [jnp.full((16,), dd, jnp.int32),
                                          biota], qpair)
                return carry

            lax.fori_loop(0, DH, qpre, 0)

            for k in range(K):
                def lbody(dd, carry, hp=hp, k=k):
                    lo, hi = carry
                    col = jnp.full((16,), hp * 32 + dd, jnp.int32)
                    qpair = plsc.load_gather(
                        qtmp, [jnp.full((16,), dd, jnp.int32), biota])
                    kpair = plsc.load_gather(knb, [rowk[k], col])
                    return (lo + _unlo(kpair) * _unlo(qpair),
                            hi + _unhi(kpair) * _unhi(qpair))

                z16 = jnp.zeros((16,), F32)
                lo, hi = lax.fori_loop(0, DH, lbody, (z16, z16))
                acch[hp][k] = lo
                acch[hp + 2][k] = hi

        for h in range(H):
            logits = [acch[h][k] * irdh
                      + (relx[k] * wx[h] + rely[k] * wy[h])
                      for k in range(K)]
            mx = logits[0]
            for k in range(1, K):
                mx = jnp.maximum(mx, logits[k])
            es = [jnp.exp(logits[k] - mx) for k in range(K)]
            ssum = es[0]
            for k in range(1, K):
                ssum = ssum + es[k]
            for k in range(K):
                attnb[pl.ds((h * K + k) * CH, CH)] = _rne_bf16(es[k] / ssum)

        for hp in range(2):
            ps_lo = [attnb[pl.ds((hp * K + k) * CH, CH)] for k in range(K)]
            ps_hi = [attnb[pl.ds(((hp + 2) * K + k) * CH, CH)]
                     for k in range(K)]

            def obody(dd, carry, hp=hp, ps_lo=ps_lo, ps_hi=ps_hi):
                col = jnp.full((16,), hp * 32 + dd, jnp.int32)
                acc_lo = jnp.zeros((16,), F32)
                acc_hi = jnp.zeros((16,), F32)
                for k in range(K):
                    vpair = plsc.load_gather(vnb, [rowk[k], col])
                    acc_lo = acc_lo + ps_lo[k] * _unlo(vpair)
                    acc_hi = acc_hi + ps_hi[k] * _unhi(vpair)
                plsc.store_scatter(ob, [biota, col], acc_lo)
                plsc.store_scatter(ob, [biota, col + 64], acc_hi)
                return carry

            lax.fori_loop(0, DH, obody, 0)

        pltpu.sync_copy(ob, o_ref.at[b, pl.ds(n0, CH)])

    fire(0, knA, vnA, semA)

    def body(i, carry):
        c0 = i * 2
        fire(c0 + 1, knB, vnB, semB)
        wait(c0, knA, vnA, semA)
        compute(c0, knA, vnA)

        @pl.when(i < NCH // 2 - 1)
        def _():
            fire(c0 + 2, knA, vnA, semA)

        wait(c0 + 1, knB, vnB, semB)
        compute(c0 + 1, knB, vnB)
        return carry

    lax.fori_loop(0, NCH // 2, body, 0)


def _sc_attn_call(kf, vf, q, posx, posy, idx2, wrel_flat, l):
    mesh = plsc.VectorSubcoreMesh(core_axis_name="c", subcore_axis_name="s")
    irdh = np.float32(1.0 / np.sqrt(DH))
    kern = functools.partial(
        pl.kernel,
        mesh=mesh,
        compiler_params=pltpu.CompilerParams(
            needs_layout_passes=False, use_tc_tiling_on_sc=False),
        out_type=jax.ShapeDtypeStruct((B, N, D), F32),
        scratch_types=[
            pltpu.VMEM((QPW * K,), jnp.int32),
            pltpu.VMEM((QPW, DP), F32),
            pltpu.VMEM((N,), F32),
            pltpu.VMEM((N,), F32),
            pltpu.VMEM((2 * H * L + 8,), F32),
            pltpu.VMEM((CH * K, DP), F32),
            pltpu.VMEM((CH * K, DP), F32),
            pltpu.VMEM((CH * K, DP), F32),
            pltpu.VMEM((CH * K, DP), F32),
            pltpu.VMEM((CH, D), F32),
            pltpu.VMEM((DH, CH), F32),
            pltpu.VMEM((H * K * CH,), F32),
            pltpu.SemaphoreType.DMA,
            pltpu.SemaphoreType.DMA,
        ],
    )(lambda *refs: _sc_attn_body(*refs, irdh=irdh, l=l))
    return kern(kf, vf, q, posx, posy, idx2, wrel_flat)


def _blk(b, nb):
    return (b, nb, 0)


def kernel(h, input_pos, Wproj, bproj, Wq, Wk, Wv, Wo, Wrel, ln1_s, ln1_b,
           W1, b1, W2, b2, ln2_s, ln2_b, Wout):
    posT = jnp.transpose(input_pos, (0, 2, 1))
    posx = input_pos[:, :, 0] + 0.0
    posy = input_pos[:, :, 1] + 0.0
    bproj2 = bproj.reshape(1, D)
    wrel_flat = jnp.pad(
        _bfc(Wrel).reshape(L * 2 * H), (0, 8)).astype(F32)
    grid = (B, NB)

    x, idx = pl.pallas_call(
        _prep_body,
        grid=grid,
        in_specs=[
            pl.BlockSpec((None, RB, DIN), _blk),
            pl.BlockSpec((None, DIN, N), lambda b, nb: (b, 0, 0)),
            pl.BlockSpec((None, RB, DIN), _blk),
            pl.BlockSpec((DIN, D), lambda b, nb: (0, 0)),
            pl.BlockSpec((1, D), lambda b, nb: (0, 0)),
        ],
        out_specs=[
            pl.BlockSpec((None, RB, D), _blk),
            pl.BlockSpec((None, RB, K), _blk),
        ],
        out_shape=[
            jax.ShapeDtypeStruct((B, N, D), F32),
            jax.ShapeDtypeStruct((B, N, K), jnp.int32),
        ],
    )(input_pos, posT, h, Wproj, bproj2)

    idx2 = idx.reshape(B, N * K)
    out = None
    for l in range(L):
        q, kf, vf = pl.pallas_call(
            _qkv_body,
            grid=grid,
            in_specs=[
                pl.BlockSpec((None, RB, D), _blk),
                pl.BlockSpec((None, 1, D), lambda b, nb, l=l: (l, 0, 0)),
                pl.BlockSpec((None, 1, D), lambda b, nb, l=l: (l, 0, 0)),
                pl.BlockSpec((None, D, D), lambda b, nb, l=l: (l, 0, 0)),
                pl.BlockSpec((None, D, D), lambda b, nb, l=l: (l, 0, 0)),
                pl.BlockSpec((None, D, D), lambda b, nb, l=l: (l, 0, 0)),
            ],
            out_specs=[
                pl.BlockSpec((None, RB, DP), _blk),
                pl.BlockSpec((None, RB, DP), _blk),
                pl.BlockSpec((None, RB, DP), _blk),
            ],
            out_shape=[
                jax.ShapeDtypeStruct((B, N, DP), F32),
                jax.ShapeDtypeStruct((B, N, DP), F32),
                jax.ShapeDtypeStruct((B, N, DP), F32),
            ],
        )(x, ln1_s.reshape(L, 1, D), ln1_b.reshape(L, 1, D), Wq, Wk, Wv)

        kf2 = kf.reshape(B * N, DP)
        vf2 = vf.reshape(B * N, DP)
        o = _sc_attn_call(kf2, vf2, q, posx, posy, idx2, wrel_flat, l)

        mlp_in_specs = [
            pl.BlockSpec((None, RB, D), _blk),
            pl.BlockSpec((None, RB, D), _blk),
            pl.BlockSpec((None, D, D), lambda b, nb, l=l: (l, 0, 0)),
            pl.BlockSpec((None, 1, D), lambda b, nb, l=l: (l, 0, 0)),
            pl.BlockSpec((None, 1, D), lambda b, nb, l=l: (l, 0, 0)),
            pl.BlockSpec((None, D, DFF), lambda b, nb, l=l: (l, 0, 0)),
            pl.BlockSpec((None, 1, DFF), lambda b, nb, l=l: (l, 0, 0)),
            pl.BlockSpec((None, DFF, D), lambda b, nb, l=l: (l, 0, 0)),
            pl.BlockSpec((None, 1, D), lambda b, nb, l=l: (l, 0, 0)),
        ]
        mlp_args = [o, x, Wo, ln2_s.reshape(L, 1, D), ln2_b.reshape(L, 1, D),
                    W1, b1.reshape(L, 1, DFF), W2, b2.reshape(L, 1, D)]
        if l == L - 1:
            mlp_in_specs.append(pl.BlockSpec((D, DOUT), lambda b, nb: (0, 0)))
            mlp_args.append(Wout)
            out = pl.pallas_call(
                _mlp_body,
                grid=grid,
                in_specs=mlp_in_specs,
                out_specs=pl.BlockSpec((None, RB, DOUT), _blk),
                out_shape=jax.ShapeDtypeStruct((B, N, DOUT), F32),
            )(*mlp_args)
        else:
            x = pl.pallas_call(
                _mlp_body,
                grid=grid,
                in_specs=mlp_in_specs,
                out_specs=pl.BlockSpec((None, RB, D), _blk),
                out_shape=jax.ShapeDtypeStruct((B, N, D), F32),
            )(*mlp_args)
    return out


# speedup vs baseline: 1.0429x; 1.0429x over previous
import functools

import jax
import jax.numpy as jnp
import numpy as np
from jax import lax
from jax.experimental import pallas as pl
from jax.experimental.pallas import tpu as pltpu
from jax.experimental.pallas import tpu_sc as plsc

B, N, DIN, D, H, DH, L, K, DFF, DOUT = 4, 2048, 2, 128, 4, 32, 3, 16, 256, 128
RB = 256
NB = N // RB
NEG = -1e30
EPS = 1e-5
BF = jnp.bfloat16
F32 = jnp.float32

NW = 32
QPW = (B * N) // NW
CH = 16
NCH = QPW // CH


def _bfc(t):
    return t.astype(BF).astype(F32)


def _mm(a, b):
    return jnp.dot(a.astype(BF), b.astype(BF), preferred_element_type=F32)


def _prep_body(pos_ref, posT_ref, h_ref, wp_ref, bp_ref,
               x_ref, idx_ref):
    pos = pos_ref[...]
    posT = posT_ref[...]
    hh = h_ref[...]
    wp = wp_ref[...]
    bp = bp_ref[...]
    x_ref[...] = (_bfc(hh[:, 0:1]) * _bfc(wp[0:1, :])
                  + _bfc(hh[:, 1:2]) * _bfc(wp[1:2, :])) + bp
    px_c, py_c = pos[:, 0:1], pos[:, 1:2]
    px_r, py_r = posT[0:1, :], posT[1:2, :]
    sq_c = px_c * px_c + py_c * py_c
    sq_r = px_r * px_r + py_r * py_r
    mm = _bfc(px_c) * _bfc(px_r) + _bfc(py_c) * _bfc(py_r)
    d2 = (sq_c + sq_r) - 2.0 * mm
    lane = jax.lax.broadcasted_iota(jnp.int32, (RB, N), 1)
    cols = []
    for _ in range(K):
        m = jnp.min(d2, axis=-1, keepdims=True)
        a2 = jnp.min(jnp.where(d2 == m, lane, jnp.int32(N)), axis=-1,
                     keepdims=True)
        d2 = jnp.where(lane == a2, 1e30, d2)
        cols.append(a2)
    idx_ref[...] = jnp.concatenate(cols, axis=1) + pl.program_id(0) * N


def _pack(x):
    bits = lax.bitcast_convert_type(_bfc(x), jnp.int32)
    lo = lax.shift_right_logical(bits[:, :64], jnp.int32(16))
    hi = bits[:, 64:] & jnp.int32(-65536)
    return lax.bitcast_convert_type(lo | hi, F32)


def _qkv_body(x_ref, s_ref, b_ref, wq_ref, wk_ref, wv_ref,
              q_ref, k_ref, v_ref):
    x = x_ref[...]
    m = jnp.mean(x, -1, keepdims=True)
    xc = x - m
    var = jnp.mean(xc * xc, -1, keepdims=True)
    xn = (xc / jnp.sqrt(var + EPS)) * s_ref[...] + b_ref[...]
    q_ref[...] = _pack(_mm(xn, wq_ref[...]))
    k_ref[...] = _pack(_mm(xn, wk_ref[...]))
    v_ref[...] = _pack(_mm(xn, wv_ref[...]))


def _mlp_body(o_ref, x_ref, wo_ref, s_ref, b_ref, w1_ref, b1_ref,
              w2_ref, b2_ref, *rest):
    if len(rest) == 2:
        wout_ref, out_ref = rest
    else:
        wout_ref, (out_ref,) = None, rest
    x = x_ref[...] + _mm(o_ref[...], wo_ref[...])
    m = jnp.mean(x, -1, keepdims=True)
    xc = x - m
    var = jnp.mean(xc * xc, -1, keepdims=True)
    xn = (xc / jnp.sqrt(var + EPS)) * s_ref[...] + b_ref[...]
    hmid = jax.nn.gelu(_mm(xn, w1_ref[...]) + b1_ref[...])
    y = x + (_mm(hmid, w2_ref[...]) + b2_ref[...])
    if wout_ref is not None:
        out_ref[...] = _mm(y, wout_ref[...])
    else:
        out_ref[...] = y


def _rne_bf16(v):
    bits = lax.bitcast_convert_type(v, jnp.int32)
    r = bits + jnp.int32(0x7FFF) + ((bits >> 16) & jnp.int32(1))
    return lax.bitcast_convert_type(r & jnp.int32(-65536), F32)


DP = D // 2


def _unlo(v):
    bits = lax.bitcast_convert_type(v, jnp.int32)
    return lax.bitcast_convert_type(bits << 16, F32)


def _unhi(v):
    bits = lax.bitcast_convert_type(v, jnp.int32)
    return lax.bitcast_convert_type(bits & jnp.int32(-65536), F32)


def _sc_attn_body(kf_ref, vf_ref, q_ref, posx_ref, posy_ref, idx_ref,
                  wrel_ref, o_ref,
                  idxst, qst, posbx, posby, wrelb,
                  knA, vnA, knB, vnB, ob, qtmp, attnb,
                  semA, semB, irdh, l):
    c_id = lax.axis_index("c")
    s_id = lax.axis_index("s")
    wid = s_id * 2 + c_id
    b = wid // 8
    nb0 = (wid % 8) * QPW
    base = b * N

    pltpu.sync_copy(posx_ref.at[b], posbx)
    pltpu.sync_copy(posy_ref.at[b], posby)
    pltpu.sync_copy(wrel_ref, wrelb)
    pltpu.sync_copy(idx_ref.at[b, pl.ds(nb0 * K, QPW * K)], idxst)
    pltpu.sync_copy(q_ref.at[b, pl.ds(nb0, QPW)], qst)

    wv0 = wrelb[pl.ds(0, 16)]
    wv1 = wrelb[pl.ds(16, 16)]

    def _wsel(i):
        return wv0[i] if i < 16 else wv1[i - 16]

    wx = [_wsel(l * 2 * H + hh) for hh in range(H)]
    wy = [_wsel(l * 2 * H + H + hh) for hh in range(H)]

    biota = lax.iota(jnp.int32, 16)
    rowk = [biota * K + k for k in range(K)]

    def fire(c, knb, vnb, sem):
        idxs = idxst.at[pl.ds(c * (CH * K), CH * K)]
        pltpu.async_copy(kf_ref.at[idxs], knb, sem)
        pltpu.async_copy(vf_ref.at[idxs], vnb, sem)

    def wait(c, knb, vnb, sem):
        idxs = idxst.at[pl.ds(c * (CH * K), CH * K)]
        pltpu.make_async_copy(kf_ref.at[idxs], knb, sem).wait()
        pltpu.make_async_copy(vf_ref.at[idxs], vnb, sem).wait()

    def compute(c, knb, vnb):
        n0 = nb0 + c * CH
        qloc = n0 + biota
        posqx = plsc.load_gather(posbx, [qloc])
        posqy = plsc.load_gather(posby, [qloc])
        relx = []
        rely = []
        for k in range(K):
            nk = plsc.load_gather(idxst, [c * (CH * K) + rowk[k]]) - base
            px = plsc.load_gather(posbx, [nk])
            py = plsc.load_gather(posby, [nk])
            relx.append(_rne_bf16(px - posqx))
            rely.append(_rne_bf16(py - posqy))

        qrow = c * CH + biota
        acch = [[None] * K for _ in range(H)]
        for hp in range(2):
            def qpre(dd, carry, hp=hp):
                col = jnp.full((16,), hp * 32 + dd, jnp.int32)
                qpair = plsc.load_gather(qst, [qrow, col])
                plsc.store_scatter(qtmp, [jnp.full((16,), dd, jnp.int32),
                                          biota], qpair)
                return carry

            lax.fori_loop(0, DH, qpre, 0)

            for k0 in range(0, K, 4):
                def lbody(dd, carry, hp=hp, k0=k0):
                    col = jnp.full((16,), hp * 32 + dd, jnp.int32)
                    qpair = plsc.load_gather(
                        qtmp, [jnp.full((16,), dd, jnp.int32), biota])
                    qlo, qhi = _unlo(qpair), _unhi(qpair)
                    out = []
                    for kk in range(4):
                        lo, hi = carry[2 * kk], carry[2 * kk + 1]
                        kpair = plsc.load_gather(knb, [rowk[k0 + kk], col])
                        out.append(lo + _unlo(kpair) * qlo)
                        out.append(hi + _unhi(kpair) * qhi)
                    return tuple(out)

                z16 = jnp.zeros((16,), F32)
                res = lax.fori_loop(0, DH, lbody, tuple([z16] * 8))
                for kk in range(4):
                    acch[hp][k0 + kk] = res[2 * kk]
                    acch[hp + 2][k0 + kk] = res[2 * kk + 1]

        for h in range(H):
            logits = [acch[h][k] * irdh
                      + (relx[k] * wx[h] + rely[k] * wy[h])
                      for k in range(K)]
            mx = logits[0]
            for k in range(1, K):
                mx = jnp.maximum(mx, logits[k])
            es = [jnp.exp(logits[k] - mx) for k in range(K)]
            ssum = es[0]
            for k in range(1, K):
                ssum = ssum + es[k]
            for k in range(K):
                attnb[pl.ds((h * K + k) * CH, CH)] = _rne_bf16(es[k] / ssum)

        for hp in range(2):
            ps_lo = [attnb[pl.ds((hp * K + k) * CH, CH)] for k in range(K)]
            ps_hi = [attnb[pl.ds(((hp + 2) * K + k) * CH, CH)]
                     for k in range(K)]

            def obody(dd, carry, hp=hp, ps_lo=ps_lo, ps_hi=ps_hi):
                col = jnp.full((16,), hp * 32 + dd, jnp.int32)
                acc_lo = jnp.zeros((16,), F32)
                acc_hi = jnp.zeros((16,), F32)
                for k in range(K):
                    vpair = plsc.load_gather(vnb, [rowk[k], col])
                    acc_lo = acc_lo + ps_lo[k] * _unlo(vpair)
                    acc_hi = acc_hi + ps_hi[k] * _unhi(vpair)
                plsc.store_scatter(ob, [biota, col], acc_lo)
                plsc.store_scatter(ob, [biota, col + 64], acc_hi)
                return carry

            lax.fori_loop(0, DH, obody, 0)

        pltpu.sync_copy(ob, o_ref.at[b, pl.ds(n0, CH)])

    fire(0, knA, vnA, semA)

    def body(i, carry):
        c0 = i * 2
        fire(c0 + 1, knB, vnB, semB)
        wait(c0, knA, vnA, semA)
        compute(c0, knA, vnA)

        @pl.when(i < NCH // 2 - 1)
        def _():
            fire(c0 + 2, knA, vnA, semA)

        wait(c0 + 1, knB, vnB, semB)
        compute(c0 + 1, knB, vnB)
        return carry

    lax.fori_loop(0, NCH // 2, body, 0)


def _sc_attn_call(kf, vf, q, posx, posy, idx2, wrel_flat, l):
    mesh = plsc.VectorSubcoreMesh(core_axis_name="c", subcore_axis_name="s")
    irdh = np.float32(1.0 / np.sqrt(DH))
    kern = functools.partial(
        pl.kernel,
        mesh=mesh,
        compiler_params=pltpu.CompilerParams(
            needs_layout_passes=False, use_tc_tiling_on_sc=False),
        out_type=jax.ShapeDtypeStruct((B, N, D), F32),
        scratch_types=[
            pltpu.VMEM((QPW * K,), jnp.int32),
            pltpu.VMEM((QPW, DP), F32),
            pltpu.VMEM((N,), F32),
            pltpu.VMEM((N,), F32),
            pltpu.VMEM((2 * H * L + 8,), F32),
            pltpu.VMEM((CH * K, DP), F32),
            pltpu.VMEM((CH * K, DP), F32),
            pltpu.VMEM((CH * K, DP), F32),
            pltpu.VMEM((CH * K, DP), F32),
            pltpu.VMEM((CH, D), F32),
            pltpu.VMEM((DH, CH), F32),
            pltpu.VMEM((H * K * CH,), F32),
            pltpu.SemaphoreType.DMA,
            pltpu.SemaphoreType.DMA,
        ],
    )(lambda *refs: _sc_attn_body(*refs, irdh=irdh, l=l))
    return kern(kf, vf, q, posx, posy, idx2, wrel_flat)


def _blk(b, nb):
    return (b, nb, 0)


def kernel(h, input_pos, Wproj, bproj, Wq, Wk, Wv, Wo, Wrel, ln1_s, ln1_b,
           W1, b1, W2, b2, ln2_s, ln2_b, Wout):
    posT = jnp.transpose(input_pos, (0, 2, 1))
    posx = input_pos[:, :, 0] + 0.0
    posy = input_pos[:, :, 1] + 0.0
    bproj2 = bproj.reshape(1, D)
    wrel_flat = jnp.pad(
        _bfc(Wrel).reshape(L * 2 * H), (0, 8)).astype(F32)
    grid = (B, NB)

    x, idx = pl.pallas_call(
        _prep_body,
        grid=grid,
        in_specs=[
            pl.BlockSpec((None, RB, DIN), _blk),
            pl.BlockSpec((None, DIN, N), lambda b, nb: (b, 0, 0)),
            pl.BlockSpec((None, RB, DIN), _blk),
            pl.BlockSpec((DIN, D), lambda b, nb: (0, 0)),
            pl.BlockSpec((1, D), lambda b, nb: (0, 0)),
        ],
        out_specs=[
            pl.BlockSpec((None, RB, D), _blk),
            pl.BlockSpec((None, RB, K), _blk),
        ],
        out_shape=[
            jax.ShapeDtypeStruct((B, N, D), F32),
            jax.ShapeDtypeStruct((B, N, K), jnp.int32),
        ],
    )(input_pos, posT, h, Wproj, bproj2)

    idx2 = idx.reshape(B, N * K)
    out = None
    for l in range(L):
        q, kf, vf = pl.pallas_call(
            _qkv_body,
            grid=grid,
            in_specs=[
                pl.BlockSpec((None, RB, D), _blk),
                pl.BlockSpec((None, 1, D), lambda b, nb, l=l: (l, 0, 0)),
                pl.BlockSpec((None, 1, D), lambda b, nb, l=l: (l, 0, 0)),
                pl.BlockSpec((None, D, D), lambda b, nb, l=l: (l, 0, 0)),
                pl.BlockSpec((None, D, D), lambda b, nb, l=l: (l, 0, 0)),
                pl.BlockSpec((None, D, D), lambda b, nb, l=l: (l, 0, 0)),
            ],
            out_specs=[
                pl.BlockSpec((None, RB, DP), _blk),
                pl.BlockSpec((None, RB, DP), _blk),
                pl.BlockSpec((None, RB, DP), _blk),
            ],
            out_shape=[
                jax.ShapeDtypeStruct((B, N, DP), F32),
                jax.ShapeDtypeStruct((B, N, DP), F32),
                jax.ShapeDtypeStruct((B, N, DP), F32),
            ],
        )(x, ln1_s.reshape(L, 1, D), ln1_b.reshape(L, 1, D), Wq, Wk, Wv)

        kf2 = kf.reshape(B * N, DP)
        vf2 = vf.reshape(B * N, DP)
        o = _sc_attn_call(kf2, vf2, q, posx, posy, idx2, wrel_flat, l)

        mlp_in_specs = [
            pl.BlockSpec((None, RB, D), _blk),
            pl.BlockSpec((None, RB, D), _blk),
            pl.BlockSpec((None, D, D), lambda b, nb, l=l: (l, 0, 0)),
            pl.BlockSpec((None, 1, D), lambda b, nb, l=l: (l, 0, 0)),
            pl.BlockSpec((None, 1, D), lambda b, nb, l=l: (l, 0, 0)),
            pl.BlockSpec((None, D, DFF), lambda b, nb, l=l: (l, 0, 0)),
            pl.BlockSpec((None, 1, DFF), lambda b, nb, l=l: (l, 0, 0)),
            pl.BlockSpec((None, DFF, D), lambda b, nb, l=l: (l, 0, 0)),
            pl.BlockSpec((None, 1, D), lambda b, nb, l=l: (l, 0, 0)),
        ]
        mlp_args = [o, x, Wo, ln2_s.reshape(L, 1, D), ln2_b.reshape(L, 1, D),
                    W1, b1.reshape(L, 1, DFF), W2, b2.reshape(L, 1, D)]
        if l == L - 1:
            mlp_in_specs.append(pl.BlockSpec((D, DOUT), lambda b, nb: (0, 0)))
            mlp_args.append(Wout)
            out = pl.pallas_call(
                _mlp_body,
                grid=grid,
                in_specs=mlp_in_specs,
                out_specs=pl.BlockSpec((None, RB, DOUT), _blk),
                out_shape=jax.ShapeDtypeStruct((B, N, DOUT), F32),
            )(*mlp_args)
        else:
            x = pl.pallas_call(
                _mlp_body,
                grid=grid,
                in_specs=mlp_in_specs,
                out_specs=pl.BlockSpec((None, RB, D), _blk),
                out_shape=jax.ShapeDtypeStruct((B, N, D), F32),
            )(*mlp_args)
    return out


# speedup vs baseline: 1.0567x; 1.0132x over previous
import functools

import jax
import jax.numpy as jnp
import numpy as np
from jax import lax
from jax.experimental import pallas as pl
from jax.experimental.pallas import tpu as pltpu
from jax.experimental.pallas import tpu_sc as plsc

B, N, DIN, D, H, DH, L, K, DFF, DOUT = 4, 2048, 2, 128, 4, 32, 3, 16, 256, 128
RB = 256
NB = N // RB
NEG = -1e30
EPS = 1e-5
BF = jnp.bfloat16
F32 = jnp.float32

NW = 32
QPW = (B * N) // NW
CH = 16
NCH = QPW // CH


def _bfc(t):
    return t.astype(BF).astype(F32)


def _mm(a, b):
    return jnp.dot(a.astype(BF), b.astype(BF), preferred_element_type=F32)


def _prep_body(pos_ref, posT_ref, h_ref, wp_ref, bp_ref,
               x_ref, idx_ref):
    pos = pos_ref[...]
    posT = posT_ref[...]
    hh = h_ref[...]
    wp = wp_ref[...]
    bp = bp_ref[...]
    x_ref[...] = (_bfc(hh[:, 0:1]) * _bfc(wp[0:1, :])
                  + _bfc(hh[:, 1:2]) * _bfc(wp[1:2, :])) + bp
    px_c, py_c = pos[:, 0:1], pos[:, 1:2]
    px_r, py_r = posT[0:1, :], posT[1:2, :]
    sq_c = px_c * px_c + py_c * py_c
    sq_r = px_r * px_r + py_r * py_r
    mm = _bfc(px_c) * _bfc(px_r) + _bfc(py_c) * _bfc(py_r)
    d2 = (sq_c + sq_r) - 2.0 * mm
    lane = jax.lax.broadcasted_iota(jnp.int32, (RB, N), 1)
    cols = []
    for _ in range(K):
        m = jnp.min(d2, axis=-1, keepdims=True)
        a2 = jnp.min(jnp.where(d2 == m, lane, jnp.int32(N)), axis=-1,
                     keepdims=True)
        d2 = jnp.where(lane == a2, 1e30, d2)
        cols.append(a2)
    idx_ref[...] = jnp.concatenate(cols, axis=1) + pl.program_id(0) * N


def _pack(x):
    bits = lax.bitcast_convert_type(_bfc(x), jnp.int32)
    lo = lax.shift_right_logical(bits[:, :64], jnp.int32(16))
    hi = bits[:, 64:] & jnp.int32(-65536)
    return lax.bitcast_convert_type(lo | hi, F32)


def _qkv_body(x_ref, s_ref, b_ref, wq_ref, wk_ref, wv_ref,
              q_ref, kv_ref):
    x = x_ref[...]
    m = jnp.mean(x, -1, keepdims=True)
    xc = x - m
    var = jnp.mean(xc * xc, -1, keepdims=True)
    xn = (xc / jnp.sqrt(var + EPS)) * s_ref[...] + b_ref[...]
    q_ref[...] = _pack(_mm(xn, wq_ref[...]))
    kv_ref[...] = jnp.concatenate(
        [_pack(_mm(xn, wk_ref[...])), _pack(_mm(xn, wv_ref[...]))], axis=-1)


def _mlp_body(o_ref, x_ref, wo_ref, s_ref, b_ref, w1_ref, b1_ref,
              w2_ref, b2_ref, *rest):
    if len(rest) == 2:
        wout_ref, out_ref = rest
    else:
        wout_ref, (out_ref,) = None, rest
    x = x_ref[...] + _mm(o_ref[...], wo_ref[...])
    m = jnp.mean(x, -1, keepdims=True)
    xc = x - m
    var = jnp.mean(xc * xc, -1, keepdims=True)
    xn = (xc / jnp.sqrt(var + EPS)) * s_ref[...] + b_ref[...]
    hmid = jax.nn.gelu(_mm(xn, w1_ref[...]) + b1_ref[...])
    y = x + (_mm(hmid, w2_ref[...]) + b2_ref[...])
    if wout_ref is not None:
        out_ref[...] = _mm(y, wout_ref[...])
    else:
        out_ref[...] = y


def _rne_bf16(v):
    bits = lax.bitcast_convert_type(v, jnp.int32)
    r = bits + jnp.int32(0x7FFF) + ((bits >> 16) & jnp.int32(1))
    return lax.bitcast_convert_type(r & jnp.int32(-65536), F32)


DP = D // 2


def _unlo(v):
    bits = lax.bitcast_convert_type(v, jnp.int32)
    return lax.bitcast_convert_type(bits << 16, F32)


def _unhi(v):
    bits = lax.bitcast_convert_type(v, jnp.int32)
    return lax.bitcast_convert_type(bits & jnp.int32(-65536), F32)


def _sc_attn_body(kv_ref, q_ref, posx_ref, posy_ref, idx_ref,
                  wrel_ref, o_ref,
                  idxst, qst, posbx, posby, wrelb,
                  kvA, kvB, ob, qtmp, attnb,
                  semA, semB, irdh, l):
    c_id = lax.axis_index("c")
    s_id = lax.axis_index("s")
    wid = s_id * 2 + c_id
    b = wid // 8
    nb0 = (wid % 8) * QPW
    base = b * N

    pltpu.sync_copy(posx_ref.at[b], posbx)
    pltpu.sync_copy(posy_ref.at[b], posby)
    pltpu.sync_copy(wrel_ref, wrelb)
    pltpu.sync_copy(idx_ref.at[b, pl.ds(nb0 * K, QPW * K)], idxst)
    pltpu.sync_copy(q_ref.at[b, pl.ds(nb0, QPW)], qst)

    wv0 = wrelb[pl.ds(0, 16)]
    wv1 = wrelb[pl.ds(16, 16)]

    def _wsel(i):
        return wv0[i] if i < 16 else wv1[i - 16]

    wx = [_wsel(l * 2 * H + hh) for hh in range(H)]
    wy = [_wsel(l * 2 * H + H + hh) for hh in range(H)]

    biota = lax.iota(jnp.int32, 16)
    rowk = [biota * K + k for k in range(K)]

    def fire(c, kvb, sem):
        idxs = idxst.at[pl.ds(c * (CH * K), CH * K)]
        pltpu.async_copy(kv_ref.at[idxs], kvb, sem)

    def wait(c, kvb, sem):
        idxs = idxst.at[pl.ds(c * (CH * K), CH * K)]
        pltpu.make_async_copy(kv_ref.at[idxs], kvb, sem).wait()

    def compute(c, knb, vnb):
        n0 = nb0 + c * CH
        qloc = n0 + biota
        posqx = plsc.load_gather(posbx, [qloc])
        posqy = plsc.load_gather(posby, [qloc])
        relx = []
        rely = []
        for k in range(K):
            nk = plsc.load_gather(idxst, [c * (CH * K) + rowk[k]]) - base
            px = plsc.load_gather(posbx, [nk])
            py = plsc.load_gather(posby, [nk])
            relx.append(_rne_bf16(px - posqx))
            rely.append(_rne_bf16(py - posqy))

        qrow = c * CH + biota
        acch = [[None] * K for _ in range(H)]
        for hp in range(2):
            def qpre(dd, carry, hp=hp):
                col = jnp.full((16,), hp * 32 + dd, jnp.int32)
                qpair = plsc.load_gather(qst, [qrow, col])
                plsc.store_scatter(qtmp, [jnp.full((16,), dd, jnp.int32),
                                          biota], qpair)
                return carry

            lax.fori_loop(0, DH, qpre, 0)

            for k0 in range(0, K, 4):
                def lbody(dd, carry, hp=hp, k0=k0):
                    col = jnp.full((16,), hp * 32 + dd, jnp.int32)
                    qpair = plsc.load_gather(
                        qtmp, [jnp.full((16,), dd, jnp.int32), biota])
                    qlo, qhi = _unlo(qpair), _unhi(qpair)
                    out = []
                    for kk in range(4):
                        lo, hi = carry[2 * kk], carry[2 * kk + 1]
                        kpair = plsc.load_gather(knb, [rowk[k0 + kk], col])
                        out.append(lo + _unlo(kpair) * qlo)
                        out.append(hi + _unhi(kpair) * qhi)
                    return tuple(out)

                z16 = jnp.zeros((16,), F32)
                res = lax.fori_loop(0, DH, lbody, tuple([z16] * 8))
                for kk in range(4):
                    acch[hp][k0 + kk] = res[2 * kk]
                    acch[hp + 2][k0 + kk] = res[2 * kk + 1]

        for h in range(H):
            logits = [acch[h][k] * irdh
                      + (relx[k] * wx[h] + rely[k] * wy[h])
                      for k in range(K)]
            mx = logits[0]
            for k in range(1, K):
                mx = jnp.maximum(mx, logits[k])
            es = [jnp.exp(logits[k] - mx) for k in range(K)]
            ssum = es[0]
            for k in range(1, K):
                ssum = ssum + es[k]
            for k in range(K):
                attnb[pl.ds((h * K + k) * CH, CH)] = _rne_bf16(es[k] / ssum)

        for hp in range(2):
            ps_lo = [attnb[pl.ds((hp * K + k) * CH, CH)] for k in range(K)]
            ps_hi = [attnb[pl.ds(((hp + 2) * K + k) * CH, CH)]
                     for k in range(K)]

            def obody(dd, carry, hp=hp, ps_lo=ps_lo, ps_hi=ps_hi):
                col = jnp.full((16,), hp * 32 + dd, jnp.int32)
                acc_lo = jnp.zeros((16,), F32)
                acc_hi = jnp.zeros((16,), F32)
                colv = col + jnp.int32(64)
                for k in range(K):
                    vpair = plsc.load_gather(vnb, [rowk[k], colv])
                    acc_lo = acc_lo + ps_lo[k] * _unlo(vpair)
                    acc_hi = acc_hi + ps_hi[k] * _unhi(vpair)
                plsc.store_scatter(ob, [biota, col], acc_lo)
                plsc.store_scatter(ob, [biota, col + 64], acc_hi)
                return carry

            lax.fori_loop(0, DH, obody, 0)

        pltpu.sync_copy(ob, o_ref.at[b, pl.ds(n0, CH)])

    fire(0, kvA, semA)

    def body(i, carry):
        c0 = i * 2
        fire(c0 + 1, kvB, semB)
        wait(c0, kvA, semA)
        compute(c0, kvA, kvA)

        @pl.when(i < NCH // 2 - 1)
        def _():
            fire(c0 + 2, kvA, semA)

        wait(c0 + 1, kvB, semB)
        compute(c0 + 1, kvB, kvB)
        return carry

    lax.fori_loop(0, NCH // 2, body, 0)


def _sc_attn_call(kv, q, posx, posy, idx2, wrel_flat, l):
    mesh = plsc.VectorSubcoreMesh(core_axis_name="c", subcore_axis_name="s")
    irdh = np.float32(1.0 / np.sqrt(DH))
    kern = functools.partial(
        pl.kernel,
        mesh=mesh,
        compiler_params=pltpu.CompilerParams(
            needs_layout_passes=False, use_tc_tiling_on_sc=False),
        out_type=jax.ShapeDtypeStruct((B, N, D), F32),
        scratch_types=[
            pltpu.VMEM((QPW * K,), jnp.int32),
            pltpu.VMEM((QPW, DP), F32),
            pltpu.VMEM((N,), F32),
            pltpu.VMEM((N,), F32),
            pltpu.VMEM((2 * H * L + 8,), F32),
            pltpu.VMEM((CH * K, D), F32),
            pltpu.VMEM((CH * K, D), F32),
            pltpu.VMEM((CH, D), F32),
            pltpu.VMEM((DH, CH), F32),
            pltpu.VMEM((H * K * CH,), F32),
            pltpu.SemaphoreType.DMA,
            pltpu.SemaphoreType.DMA,
        ],
    )(lambda *refs: _sc_attn_body(*refs, irdh=irdh, l=l))
    return kern(kv, q, posx, posy, idx2, wrel_flat)


def _blk(b, nb):
    return (b, nb, 0)


def kernel(h, input_pos, Wproj, bproj, Wq, Wk, Wv, Wo, Wrel, ln1_s, ln1_b,
           W1, b1, W2, b2, ln2_s, ln2_b, Wout):
    posT = jnp.transpose(input_pos, (0, 2, 1))
    posx = input_pos[:, :, 0] + 0.0
    posy = input_pos[:, :, 1] + 0.0
    bproj2 = bproj.reshape(1, D)
    wrel_flat = jnp.pad(
        _bfc(Wrel).reshape(L * 2 * H), (0, 8)).astype(F32)
    grid = (B, NB)

    x, idx = pl.pallas_call(
        _prep_body,
        grid=grid,
        in_specs=[
            pl.BlockSpec((None, RB, DIN), _blk),
            pl.BlockSpec((None, DIN, N), lambda b, nb: (b, 0, 0)),
            pl.BlockSpec((None, RB, DIN), _blk),
            pl.BlockSpec((DIN, D), lambda b, nb: (0, 0)),
            pl.BlockSpec((1, D), lambda b, nb: (0, 0)),
        ],
        out_specs=[
            pl.BlockSpec((None, RB, D), _blk),
            pl.BlockSpec((None, RB, K), _blk),
        ],
        out_shape=[
            jax.ShapeDtypeStruct((B, N, D), F32),
            jax.ShapeDtypeStruct((B, N, K), jnp.int32),
        ],
    )(input_pos, posT, h, Wproj, bproj2)

    idx2 = idx.reshape(B, N * K)
    out = None
    for l in range(L):
        q, kvf = pl.pallas_call(
            _qkv_body,
            grid=grid,
            in_specs=[
                pl.BlockSpec((None, RB, D), _blk),
                pl.BlockSpec((None, 1, D), lambda b, nb, l=l: (l, 0, 0)),
                pl.BlockSpec((None, 1, D), lambda b, nb, l=l: (l, 0, 0)),
                pl.BlockSpec((None, D, D), lambda b, nb, l=l: (l, 0, 0)),
                pl.BlockSpec((None, D, D), lambda b, nb, l=l: (l, 0, 0)),
                pl.BlockSpec((None, D, D), lambda b, nb, l=l: (l, 0, 0)),
            ],
            out_specs=[
                pl.BlockSpec((None, RB, DP), _blk),
                pl.BlockSpec((None, RB, D), _blk),
            ],
            out_shape=[
                jax.ShapeDtypeStruct((B, N, DP), F32),
                jax.ShapeDtypeStruct((B, N, D), F32),
            ],
        )(x, ln1_s.reshape(L, 1, D), ln1_b.reshape(L, 1, D), Wq, Wk, Wv)

        kv2 = kvf.reshape(B * N, D)
        o = _sc_attn_call(kv2, q, posx, posy, idx2, wrel_flat, l)

        mlp_in_specs = [
            pl.BlockSpec((None, RB, D), _blk),
            pl.BlockSpec((None, RB, D), _blk),
            pl.BlockSpec((None, D, D), lambda b, nb, l=l: (l, 0, 0)),
            pl.BlockSpec((None, 1, D), lambda b, nb, l=l: (l, 0, 0)),
            pl.BlockSpec((None, 1, D), lambda b, nb, l=l: (l, 0, 0)),
            pl.BlockSpec((None, D, DFF), lambda b, nb, l=l: (l, 0, 0)),
            pl.BlockSpec((None, 1, DFF), lambda b, nb, l=l: (l, 0, 0)),
            pl.BlockSpec((None, DFF, D), lambda b, nb, l=l: (l, 0, 0)),
            pl.BlockSpec((None, 1, D), lambda b, nb, l=l: (l, 0, 0)),
        ]
        mlp_args = [o, x, Wo, ln2_s.reshape(L, 1, D), ln2_b.reshape(L, 1, D),
                    W1, b1.reshape(L, 1, DFF), W2, b2.reshape(L, 1, D)]
        if l == L - 1:
            mlp_in_specs.append(pl.BlockSpec((D, DOUT), lambda b, nb: (0, 0)))
            mlp_args.append(Wout)
            out = pl.pallas_call(
                _mlp_body,
                grid=grid,
                in_specs=mlp_in_specs,
                out_specs=pl.BlockSpec((None, RB, DOUT), _blk),
                out_shape=jax.ShapeDtypeStruct((B, N, DOUT), F32),
            )(*mlp_args)
        else:
            x = pl.pallas_call(
                _mlp_body,
                grid=grid,
                in_specs=mlp_in_specs,
                out_specs=pl.BlockSpec((None, RB, D), _blk),
                out_shape=jax.ShapeDtypeStruct((B, N, D), F32),
            )(*mlp_args)
    return out
